# baseline (speedup 1.0000x reference)
import functools
import math

import jax
import jax.numpy as jnp
import numpy as np
from jax import lax
from jax.experimental import pallas as pl
from jax.experimental.pallas import tpu as pltpu
from jax.experimental.pallas import tpu_sc as plsc

_N_NODES = 10000
_N_EDGES = 160000
_D_IN = 32
_D_OUT = 32
_NUM_BASIS = 10
_H = 100
_R_DIM = 384
_MIN_R, _MAX_R = 0.7, 3.2
_STEP = (_MAX_R - _MIN_R) / (_NUM_BASIS - 1)

_EBLK = 1280
_SUBW = 256
_NSUB = _EBLK // _SUBW
_N_EBLK = _N_EDGES // _EBLK


def _w3j_c(j1, j2, j3, m1, m2, m3):
    if m1 + m2 + m3 != 0 or not (abs(j1 - j2) <= j3 <= j1 + j2):
        return 0.0
    f = math.factorial
    delta = math.sqrt(f(j1 + j2 - j3) * f(j1 - j2 + j3) * f(-j1 + j2 + j3) / f(j1 + j2 + j3 + 1))
    pref = delta * math.sqrt(f(j1 + m1) * f(j1 - m1) * f(j2 + m2) * f(j2 - m2) * f(j3 + m3) * f(j3 - m3))
    kmin = max(0, j2 - j3 - m1, j1 - j3 + m2)
    kmax = min(j1 + j2 - j3, j1 - m1, j2 + m2)
    s = 0.0
    for k in range(kmin, kmax + 1):
        s += (-1.0) ** k / (
            f(k) * f(j1 + j2 - j3 - k) * f(j1 - m1 - k) * f(j2 + m2 - k)
            * f(j3 - j2 + m1 + k) * f(j3 - j1 - m2 + k))
    return ((-1.0) ** (j1 - j2 - m3)) * pref * s


def _u_real(l):
    U = np.zeros((2 * l + 1, 2 * l + 1), dtype=complex)
    s2 = math.sqrt(2.0)
    for m in range(-l, l + 1):
        if m == 0:
            U[l, l] = 1.0
        elif m > 0:
            U[l + m, l - m] = 1.0 / s2
            U[l + m, l + m] = ((-1.0) ** m) / s2
        else:
            a = -m
            U[l + m, l - a] = 1j / s2
            U[l + m, l + a] = -1j * ((-1.0) ** a) / s2
    return U


def _wigner_3j_real(l1, l2, l3):
    C = np.zeros((2 * l1 + 1, 2 * l2 + 1, 2 * l3 + 1), dtype=complex)
    for m1 in range(-l1, l1 + 1):
        for m2 in range(-l2, l2 + 1):
            m3 = -(m1 + m2)
            if -l3 <= m3 <= l3:
                C[m1 + l1, m2 + l2, m3 + l3] = _w3j_c(l1, l2, l3, m1, m2, m3)
    T = np.einsum('am,bn,co,mno->abc', _u_real(l1), _u_real(l2), _u_real(l3), C)
    flat = T.reshape(-1)
    k = int(np.argmax(np.abs(flat)))
    if abs(flat[k]) > 0:
        ph = flat[k] / abs(flat[k])
        T = T * np.conj(ph)
    return np.real(T).astype(np.float64)


_C011 = _wigner_3j_real(0, 1, 1)
_C101 = _wigner_3j_real(1, 0, 1)
_C110 = _wigner_3j_real(1, 1, 0)
_C111 = _wigner_3j_real(1, 1, 1)
_C112 = _wigner_3j_real(1, 1, 2)

_SQ4PI = math.sqrt(4 * math.pi)
_NORM0 = _SQ4PI * math.sqrt(1.0) / math.sqrt(8 * 1 + 8 * 1)
_NORM1 = _SQ4PI * math.sqrt(3.0) / math.sqrt(8 * 1 + 8 * 3)

_DELTA3 = float(_C011[0, 0, 0])
_EPS = float(abs(_C111[0, 1, 2]))
_EPS_TERMS = [[(b, c, float(np.sign(_C111[a, b, c])))
               for b in range(3) for c in range(3)
               if abs(_C111[a, b, c]) > 1e-12] for a in range(3)]
_C112_TERMS = [[[(c, float(_C112[a, b, c])) for c in range(5)
                 if abs(_C112[a, b, c]) > 1e-12] for b in range(3)]
               for a in range(3)]

_INV_SQRT_H = 1.0 / math.sqrt(_H)
_INV_SQRT_B = 1.0 / math.sqrt(_NUM_BASIS)


def _build_w3_perm_scale():
    perm = np.zeros(_R_DIM, dtype=np.int64)
    scale = np.zeros(_R_DIM, dtype=np.float64)
    for v in range(8):
        for u in range(8):
            perm[v * 8 + u] = u * 8 + v
            scale[v * 8 + u] = _NORM0
            perm[64 + v * 8 + u] = 64 + u * 8 + v
            scale[64 + v * 8 + u] = _NORM0 * _DELTA3
            perm[128 + v * 8 + u] = 128 + u * 8 + v
            scale[128 + v * 8 + u] = _NORM1 * _DELTA3
            for fi in range(3):
                perm[192 + (v * 3 + fi) * 8 + u] = 192 + (u * 8 + v) * 3 + fi
                scale[192 + (v * 3 + fi) * 8 + u] = _NORM1 * (
                    _DELTA3 if fi == 0 else (_EPS if fi == 1 else 1.0))
    return perm, (scale * _INV_SQRT_H).astype(np.float32)


_W3_PERM, _W3_SCALE = _build_w3_perm_scale()

_F_PERM = np.concatenate([
    np.arange(8),
    np.array([8 + v * 3 + b for b in range(3) for v in range(8)]),
]).astype(np.int32)

_MSG_PERM = np.concatenate([
    np.arange(8),
    np.array([8 + a * 8 + u for u in range(8) for a in range(3)]),
]).astype(np.int32)


def _silu(z):
    return z * (1.0 / (1.0 + jnp.exp(-z)))


def _tree_sum(xs):
    xs = list(xs)
    while len(xs) > 1:
        nxt = [a + b for a, b in zip(xs[0::2], xs[1::2])]
        if len(xs) % 2:
            nxt.append(xs[-1])
        xs = nxt
    return xs[0]


def _tc_fused_body(radii_ref, yt_ref, ft_ref, w0_ref, w1_ref, w2_ref, w3_ref,
                   out_ref):
    f32 = jnp.float32
    dn = (((1,), (0,)), ((), ()))
    rrow_all = radii_ref[...]
    ft_all = ft_ref[...].T
    yt_all = yt_ref[...]
    centers = (_MIN_R + _STEP * lax.broadcasted_iota(
        jnp.int32, (_NUM_BASIS, _SUBW), 0).astype(f32))

    for sub in range(_NSUB):
        ls = slice(sub * _SUBW, (sub + 1) * _SUBW)
        rrow = rrow_all[:, ls]
        ft = ft_all[:, ls]
        yt = yt_all[:, ls]

        t = (jnp.broadcast_to(rrow, (_NUM_BASIS, _SUBW)) - centers) * (1.0 / _STEP)
        b = jnp.exp(-(t * t))
        h = _silu(lax.dot_general(w0_ref[...], b, dn, preferred_element_type=f32)
                  * _INV_SQRT_B)
        h = _silu(lax.dot_general(w1_ref[...], h, dn, preferred_element_type=f32)
                  * _INV_SQRT_H)
        h = _silu(lax.dot_general(w2_ref[...], h, dn, preferred_element_type=f32)
                  * _INV_SQRT_H)
        rt = lax.dot_general(w3_ref[...], h, dn, preferred_element_type=f32)

        f0 = ft[0:8]
        f1 = [ft[8 + bb * 8: 16 + bb * 8] for bb in range(3)]
        y0 = yt[0:1]
        y1 = [yt[1 + c: 2 + c] for c in range(3)]
        y2 = [yt[4 + c: 5 + c] for c in range(5)]

        pa = f0 * y0
        pb = f1[0] * y1[0] + (f1[1] * y1[1] + f1[2] * y1[2])
        acc0 = _tree_sum(
            [rt[v * 8: v * 8 + 8] * pa[v: v + 1] for v in range(8)]
            + [rt[64 + v * 8: 72 + v * 8] * pb[v: v + 1] for v in range(8)])
        accs = [acc0]

        for a in range(3):
            pc = f0 * y1[a]
            pd0 = f1[a] * y0
            pd1 = _tree_sum([f1[bb] * (s * y1[c])
                             for bb, c, s in _EPS_TERMS[a]])
            pd2 = _tree_sum([
                f1[bb] * _tree_sum([coeff * y2[c]
                                    for c, coeff in _C112_TERMS[a][bb]])
                for bb in range(3)])
            terms = []
            for v in range(8):
                base = 192 + v * 24
                terms += [
                    rt[128 + v * 8: 136 + v * 8] * pc[v: v + 1],
                    rt[base: base + 8] * pd0[v: v + 1],
                    rt[base + 8: base + 16] * pd1[v: v + 1],
                    rt[base + 16: base + 24] * pd2[v: v + 1],
                ]
            accs.append(_tree_sum(terms))
        out_ref[ls, :] = jnp.concatenate(accs, axis=0).T


def _tc_fused(radii2d, yt, fe, w0t, w1t, w2t, w3pt, *, interpret=False):
    const = lambda shape: pl.BlockSpec(shape, lambda i: (0, 0))
    eb = lambda rows: pl.BlockSpec((rows, _EBLK), lambda i: (0, i))
    erow = pl.BlockSpec((_EBLK, _D_IN), lambda i: (i, 0))
    return pl.pallas_call(
        _tc_fused_body,
        grid=(_N_EBLK,),
        in_specs=[
            eb(1),
            eb(9),
            erow,
            const((_H, _NUM_BASIS)),
            const((_H, _H)),
            const((_H, _H)),
            const((_R_DIM, _H)),
        ],
        out_specs=pl.BlockSpec((_EBLK, _D_OUT), lambda i: (i, 0)),
        out_shape=jax.ShapeDtypeStruct((_E_PAD, _D_OUT), jnp.float32),
        interpret=interpret,
    )(radii2d, yt, fe, w0t, w1t, w2t, w3pt)


_NW = 32
_CHUNK = 128
_CHUNKS_PER_W = 40
_E_PAD = _NW * _CHUNKS_PER_W * _CHUNK
_E_PER_W = _E_PAD // _NW
_IDX_ROWS = _E_PAD // _CHUNK
_N_ACC = 10240
_NODES_PER_W = _N_ACC // 16


_SC_PARAMS = pltpu.CompilerParams(use_tc_tiling_on_sc=False)
_NBUF = 4


def _sc_gather(xp, src2d):
    mesh = plsc.VectorSubcoreMesh(core_axis_name="c", subcore_axis_name="s")

    @functools.partial(
        pl.kernel,
        out_type=jax.ShapeDtypeStruct((_E_PAD, _D_IN), jnp.float32),
        mesh=mesh,
        scratch_types=[
            pltpu.VMEM((_CHUNKS_PER_W, _CHUNK), jnp.int32),
            pltpu.VMEM((_NBUF, _CHUNK, _D_IN), jnp.float32),
        ] + [pltpu.SemaphoreType.DMA] * _NBUF,
        compiler_params=_SC_PARAMS,
    )
    def k(x_hbm, idx_hbm, out_hbm, idx_v, bufs, *sems):
        wid = lax.axis_index("c") * 16 + lax.axis_index("s")
        pltpu.sync_copy(
            idx_hbm.at[pl.ds(wid * _CHUNKS_PER_W, _CHUNKS_PER_W)], idx_v)

        def body(t, carry):
            hs = []
            for p in range(_NBUF):
                j = t * _NBUF + p
                hs.append(pltpu.async_copy(
                    x_hbm.at[idx_v.at[j]], bufs.at[p], sems[p]))
            for p in range(_NBUF):
                j = t * _NBUF + p
                hs[p].wait()
                pltpu.sync_copy(
                    bufs.at[p],
                    out_hbm.at[pl.ds(wid * _E_PER_W + j * _CHUNK, _CHUNK)])
            return carry

        lax.fori_loop(0, _CHUNKS_PER_W // _NBUF, body, 0)

    return k(xp, src2d)


def _sc_scatter(msg, dst2d, zrows):
    mesh = plsc.VectorSubcoreMesh(core_axis_name="c", subcore_axis_name="s")

    @functools.partial(
        pl.kernel,
        out_type=jax.ShapeDtypeStruct((2, _N_ACC, _D_OUT), jnp.float32),
        mesh=mesh,
        scratch_types=[
            pltpu.VMEM((_CHUNKS_PER_W, _CHUNK), jnp.int32),
            pltpu.VMEM((_NBUF, _CHUNK, _D_OUT), jnp.float32),
            pltpu.VMEM_SHARED((_N_ACC, _D_OUT), jnp.float32),
        ] + [pltpu.SemaphoreType.DMA] * _NBUF,
        compiler_params=_SC_PARAMS,
    )
    def k(msg_hbm, idx_hbm, z_hbm, out_hbm, idx_v, bufs, acc_sh, *sems):
        c = lax.axis_index("c")
        s = lax.axis_index("s")
        wid = c * 16 + s
        pltpu.sync_copy(z_hbm, acc_sh.at[pl.ds(s * _NODES_PER_W, _NODES_PER_W)])
        pltpu.sync_copy(
            idx_hbm.at[pl.ds(wid * _CHUNKS_PER_W, _CHUNKS_PER_W)], idx_v)
        plsc.subcore_barrier()

        def body(t, carry):
            hs = []
            for p in range(_NBUF):
                j = t * _NBUF + p
                hs.append(pltpu.async_copy(
                    msg_hbm.at[pl.ds(wid * _E_PER_W + j * _CHUNK, _CHUNK)],
                    bufs.at[p], sems[p]))
            for p in range(_NBUF):
                j = t * _NBUF + p
                hs[p].wait()
                pltpu.sync_copy(bufs.at[p], acc_sh.at[idx_v.at[j]], add=True)
            return carry

        lax.fori_loop(0, _CHUNKS_PER_W // _NBUF, body, 0)
        plsc.subcore_barrier()
        pltpu.sync_copy(
            acc_sh.at[pl.ds(s * _NODES_PER_W, _NODES_PER_W)],
            out_hbm.at[c, pl.ds(s * _NODES_PER_W, _NODES_PER_W)])

    return k(msg, dst2d, zrows)


@jax.jit
def kernel(x, edge_index, radii, rsh, W0, W1, W2, W3):
    src = edge_index[0].astype(jnp.int32)
    dst = edge_index[1].astype(jnp.int32)

    npad = _E_PAD - _N_EDGES
    xp = x[:, _F_PERM]
    src_pad = jnp.concatenate([src, jnp.zeros((npad,), jnp.int32)])
    f_e = _sc_gather(xp, src_pad.reshape(_IDX_ROWS, _CHUNK))
    yt = rsh.T
    radii2d = radii.reshape(1, _N_EDGES)
    w3pt = (W3[:, _W3_PERM] * _W3_SCALE[None, :]).T

    msg_pad = _tc_fused(radii2d, yt, f_e, W0.T, W1.T, W2.T, w3pt)
    dst_pad = jnp.concatenate([dst, jnp.full((npad,), _N_NODES, jnp.int32)])

    zrows = jnp.zeros((_NODES_PER_W, _D_OUT), jnp.float32)
    parts = _sc_scatter(msg_pad, dst_pad.reshape(_IDX_ROWS, _CHUNK), zrows)
    return (parts[0, :_N_NODES] + parts[1, :_N_NODES])[:, _MSG_PERM]


# speedup vs baseline: 1.1086x; 1.1086x over previous
import functools
import math

import jax
import jax.numpy as jnp
import numpy as np
from jax import lax
from jax.experimental import pallas as pl
from jax.experimental.pallas import tpu as pltpu
from jax.experimental.pallas import tpu_sc as plsc

_N_NODES = 10000
_N_EDGES = 160000
_D_IN = 32
_D_OUT = 32
_NUM_BASIS = 10
_H = 100
_R_DIM = 384
_MIN_R, _MAX_R = 0.7, 3.2
_STEP = (_MAX_R - _MIN_R) / (_NUM_BASIS - 1)

_EBLK = 1280
_SUBW = 640
_NSUB = _EBLK // _SUBW
_N_EBLK = _N_EDGES // _EBLK


def _w3j_c(j1, j2, j3, m1, m2, m3):
    if m1 + m2 + m3 != 0 or not (abs(j1 - j2) <= j3 <= j1 + j2):
        return 0.0
    f = math.factorial
    delta = math.sqrt(f(j1 + j2 - j3) * f(j1 - j2 + j3) * f(-j1 + j2 + j3) / f(j1 + j2 + j3 + 1))
    pref = delta * math.sqrt(f(j1 + m1) * f(j1 - m1) * f(j2 + m2) * f(j2 - m2) * f(j3 + m3) * f(j3 - m3))
    kmin = max(0, j2 - j3 - m1, j1 - j3 + m2)
    kmax = min(j1 + j2 - j3, j1 - m1, j2 + m2)
    s = 0.0
    for k in range(kmin, kmax + 1):
        s += (-1.0) ** k / (
            f(k) * f(j1 + j2 - j3 - k) * f(j1 - m1 - k) * f(j2 + m2 - k)
            * f(j3 - j2 + m1 + k) * f(j3 - j1 - m2 + k))
    return ((-1.0) ** (j1 - j2 - m3)) * pref * s


def _u_real(l):
    U = np.zeros((2 * l + 1, 2 * l + 1), dtype=complex)
    s2 = math.sqrt(2.0)
    for m in range(-l, l + 1):
        if m == 0:
            U[l, l] = 1.0
        elif m > 0:
            U[l + m, l - m] = 1.0 / s2
            U[l + m, l + m] = ((-1.0) ** m) / s2
        else:
            a = -m
            U[l + m, l - a] = 1j / s2
            U[l + m, l + a] = -1j * ((-1.0) ** a) / s2
    return U


def _wigner_3j_real(l1, l2, l3):
    C = np.zeros((2 * l1 + 1, 2 * l2 + 1, 2 * l3 + 1), dtype=complex)
    for m1 in range(-l1, l1 + 1):
        for m2 in range(-l2, l2 + 1):
            m3 = -(m1 + m2)
            if -l3 <= m3 <= l3:
                C[m1 + l1, m2 + l2, m3 + l3] = _w3j_c(l1, l2, l3, m1, m2, m3)
    T = np.einsum('am,bn,co,mno->abc', _u_real(l1), _u_real(l2), _u_real(l3), C)
    flat = T.reshape(-1)
    k = int(np.argmax(np.abs(flat)))
    if abs(flat[k]) > 0:
        ph = flat[k] / abs(flat[k])
        T = T * np.conj(ph)
    return np.real(T).astype(np.float64)


_C011 = _wigner_3j_real(0, 1, 1)
_C101 = _wigner_3j_real(1, 0, 1)
_C110 = _wigner_3j_real(1, 1, 0)
_C111 = _wigner_3j_real(1, 1, 1)
_C112 = _wigner_3j_real(1, 1, 2)

_SQ4PI = math.sqrt(4 * math.pi)
_NORM0 = _SQ4PI * math.sqrt(1.0) / math.sqrt(8 * 1 + 8 * 1)
_NORM1 = _SQ4PI * math.sqrt(3.0) / math.sqrt(8 * 1 + 8 * 3)

_DELTA3 = float(_C011[0, 0, 0])
_EPS = float(abs(_C111[0, 1, 2]))
_EPS_TERMS = [[(b, c, float(np.sign(_C111[a, b, c])))
               for b in range(3) for c in range(3)
               if abs(_C111[a, b, c]) > 1e-12] for a in range(3)]
_C112_TERMS = [[[(c, float(_C112[a, b, c])) for c in range(5)
                 if abs(_C112[a, b, c]) > 1e-12] for b in range(3)]
               for a in range(3)]

_INV_SQRT_H = 1.0 / math.sqrt(_H)
_INV_SQRT_B = 1.0 / math.sqrt(_NUM_BASIS)


def _build_w3_perm_scale():
    perm = np.zeros(_R_DIM, dtype=np.int64)
    scale = np.zeros(_R_DIM, dtype=np.float64)
    for v in range(8):
        for u in range(8):
            perm[v * 8 + u] = u * 8 + v
            scale[v * 8 + u] = _NORM0
            perm[64 + v * 8 + u] = 64 + u * 8 + v
            scale[64 + v * 8 + u] = _NORM0 * _DELTA3
            perm[128 + v * 8 + u] = 128 + u * 8 + v
            scale[128 + v * 8 + u] = _NORM1 * _DELTA3
            for fi in range(3):
                perm[192 + (v * 3 + fi) * 8 + u] = 192 + (u * 8 + v) * 3 + fi
                scale[192 + (v * 3 + fi) * 8 + u] = _NORM1 * (
                    _DELTA3 if fi == 0 else (_EPS if fi == 1 else 1.0))
    return perm, (scale * _INV_SQRT_H).astype(np.float32)


_W3_PERM, _W3_SCALE = _build_w3_perm_scale()

_F_PERM = np.concatenate([
    np.arange(8),
    np.array([8 + v * 3 + b for b in range(3) for v in range(8)]),
]).astype(np.int32)

_MSG_PERM = np.concatenate([
    np.arange(8),
    np.array([8 + a * 8 + u for u in range(8) for a in range(3)]),
]).astype(np.int32)


def _silu(z):
    return z * (1.0 / (1.0 + jnp.exp(-z)))


def _tree_sum(xs):
    xs = list(xs)
    while len(xs) > 1:
        nxt = [a + b for a, b in zip(xs[0::2], xs[1::2])]
        if len(xs) % 2:
            nxt.append(xs[-1])
        xs = nxt
    return xs[0]


def _tc_fused_body(radii_ref, yt_ref, ft_ref, w0_ref, w1_ref, w2_ref, w3_ref,
                   out_ref):
    f32 = jnp.float32
    dn = (((1,), (0,)), ((), ()))
    rrow_all = radii_ref[...]
    ft_all = ft_ref[...].T
    yt_all = yt_ref[...]
    centers = (_MIN_R + _STEP * lax.broadcasted_iota(
        jnp.int32, (_NUM_BASIS, _SUBW), 0).astype(f32))

    for sub in range(_NSUB):
        ls = slice(sub * _SUBW, (sub + 1) * _SUBW)
        rrow = rrow_all[:, ls]
        ft = ft_all[:, ls]
        yt = yt_all[:, ls]

        t = (jnp.broadcast_to(rrow, (_NUM_BASIS, _SUBW)) - centers) * (1.0 / _STEP)
        b = jnp.exp(-(t * t))
        h = _silu(lax.dot_general(w0_ref[...], b, dn, preferred_element_type=f32)
                  * _INV_SQRT_B)
        h = _silu(lax.dot_general(w1_ref[...], h, dn, preferred_element_type=f32)
                  * _INV_SQRT_H)
        h = _silu(lax.dot_general(w2_ref[...], h, dn, preferred_element_type=f32)
                  * _INV_SQRT_H)
        rt = lax.dot_general(w3_ref[...], h, dn, preferred_element_type=f32)

        f0 = ft[0:8]
        f1 = [ft[8 + bb * 8: 16 + bb * 8] for bb in range(3)]
        y0 = yt[0:1]
        y1 = [yt[1 + c: 2 + c] for c in range(3)]
        y2 = [yt[4 + c: 5 + c] for c in range(5)]

        pa = f0 * y0
        pb = f1[0] * y1[0] + (f1[1] * y1[1] + f1[2] * y1[2])
        acc0 = _tree_sum(
            [rt[v * 8: v * 8 + 8] * pa[v: v + 1] for v in range(8)]
            + [rt[64 + v * 8: 72 + v * 8] * pb[v: v + 1] for v in range(8)])
        accs = [acc0]

        for a in range(3):
            pc = f0 * y1[a]
            pd0 = f1[a] * y0
            pd1 = _tree_sum([f1[bb] * (s * y1[c])
                             for bb, c, s in _EPS_TERMS[a]])
            pd2 = _tree_sum([
                f1[bb] * _tree_sum([coeff * y2[c]
                                    for c, coeff in _C112_TERMS[a][bb]])
                for bb in range(3)])
            terms = []
            for v in range(8):
                base = 192 + v * 24
                terms += [
                    rt[128 + v * 8: 136 + v * 8] * pc[v: v + 1],
                    rt[base: base + 8] * pd0[v: v + 1],
                    rt[base + 8: base + 16] * pd1[v: v + 1],
                    rt[base + 16: base + 24] * pd2[v: v + 1],
                ]
            accs.append(_tree_sum(terms))
        out_ref[ls, :] = jnp.concatenate(accs, axis=0).T


def _tc_fused(radii2d, yt, fe, w0t, w1t, w2t, w3pt, *, interpret=False):
    const = lambda shape: pl.BlockSpec(shape, lambda i: (0, 0))
    eb = lambda rows: pl.BlockSpec((rows, _EBLK), lambda i: (0, i))
    erow = pl.BlockSpec((_EBLK, _D_IN), lambda i: (i, 0))
    return pl.pallas_call(
        _tc_fused_body,
        grid=(_N_EBLK,),
        in_specs=[
            eb(1),
            eb(9),
            erow,
            const((_H, _NUM_BASIS)),
            const((_H, _H)),
            const((_H, _H)),
            const((_R_DIM, _H)),
        ],
        out_specs=pl.BlockSpec((_EBLK, _D_OUT), lambda i: (i, 0)),
        out_shape=jax.ShapeDtypeStruct((_E_PAD, _D_OUT), jnp.float32),
        interpret=interpret,
    )(radii2d, yt, fe, w0t, w1t, w2t, w3pt)


_NW = 32
_CHUNK = 128
_CHUNKS_PER_W = 40
_E_PAD = _NW * _CHUNKS_PER_W * _CHUNK
_E_PER_W = _E_PAD // _NW
_IDX_ROWS = _E_PAD // _CHUNK
_N_ACC = 10240
_NODES_PER_W = _N_ACC // 16


_SC_PARAMS = pltpu.CompilerParams(use_tc_tiling_on_sc=False)
_NBUF = 4


def _sc_gather(xp, src2d):
    mesh = plsc.VectorSubcoreMesh(core_axis_name="c", subcore_axis_name="s")

    @functools.partial(
        pl.kernel,
        out_type=jax.ShapeDtypeStruct((_E_PAD, _D_IN), jnp.float32),
        mesh=mesh,
        scratch_types=[
            pltpu.VMEM((_CHUNKS_PER_W, _CHUNK), jnp.int32),
            pltpu.VMEM((_NBUF, _CHUNK, _D_IN), jnp.float32),
        ] + [pltpu.SemaphoreType.DMA] * _NBUF,
        compiler_params=_SC_PARAMS,
    )
    def k(x_hbm, idx_hbm, out_hbm, idx_v, bufs, *sems):
        wid = lax.axis_index("c") * 16 + lax.axis_index("s")
        pltpu.sync_copy(
            idx_hbm.at[pl.ds(wid * _CHUNKS_PER_W, _CHUNKS_PER_W)], idx_v)

        def body(t, carry):
            hs = []
            for p in range(_NBUF):
                j = t * _NBUF + p
                hs.append(pltpu.async_copy(
                    x_hbm.at[idx_v.at[j]], bufs.at[p], sems[p]))
            for p in range(_NBUF):
                j = t * _NBUF + p
                hs[p].wait()
                pltpu.sync_copy(
                    bufs.at[p],
                    out_hbm.at[pl.ds(wid * _E_PER_W + j * _CHUNK, _CHUNK)])
            return carry

        lax.fori_loop(0, _CHUNKS_PER_W // _NBUF, body, 0)

    return k(xp, src2d)


def _sc_scatter(msg, dst2d, zrows):
    mesh = plsc.VectorSubcoreMesh(core_axis_name="c", subcore_axis_name="s")

    @functools.partial(
        pl.kernel,
        out_type=jax.ShapeDtypeStruct((2, _N_ACC, _D_OUT), jnp.float32),
        mesh=mesh,
        scratch_types=[
            pltpu.VMEM((_CHUNKS_PER_W, _CHUNK), jnp.int32),
            pltpu.VMEM((_NBUF, _CHUNK, _D_OUT), jnp.float32),
            pltpu.VMEM_SHARED((_N_ACC, _D_OUT), jnp.float32),
        ] + [pltpu.SemaphoreType.DMA] * _NBUF,
        compiler_params=_SC_PARAMS,
    )
    def k(msg_hbm, idx_hbm, z_hbm, out_hbm, idx_v, bufs, acc_sh, *sems):
        c = lax.axis_index("c")
        s = lax.axis_index("s")
        wid = c * 16 + s
        pltpu.sync_copy(z_hbm, acc_sh.at[pl.ds(s * _NODES_PER_W, _NODES_PER_W)])
        pltpu.sync_copy(
            idx_hbm.at[pl.ds(wid * _CHUNKS_PER_W, _CHUNKS_PER_W)], idx_v)
        plsc.subcore_barrier()

        def body(t, carry):
            hs = []
            for p in range(_NBUF):
                j = t * _NBUF + p
                hs.append(pltpu.async_copy(
                    msg_hbm.at[pl.ds(wid * _E_PER_W + j * _CHUNK, _CHUNK)],
                    bufs.at[p], sems[p]))
            for p in range(_NBUF):
                j = t * _NBUF + p
                hs[p].wait()
                pltpu.sync_copy(bufs.at[p], acc_sh.at[idx_v.at[j]], add=True)
            return carry

        lax.fori_loop(0, _CHUNKS_PER_W // _NBUF, body, 0)
        plsc.subcore_barrier()
        pltpu.sync_copy(
            acc_sh.at[pl.ds(s * _NODES_PER_W, _NODES_PER_W)],
            out_hbm.at[c, pl.ds(s * _NODES_PER_W, _NODES_PER_W)])

    return k(msg, dst2d, zrows)


@jax.jit
def kernel(x, edge_index, radii, rsh, W0, W1, W2, W3):
    src = edge_index[0].astype(jnp.int32)
    dst = edge_index[1].astype(jnp.int32)

    npad = _E_PAD - _N_EDGES
    xp = x[:, _F_PERM]
    src_pad = jnp.concatenate([src, jnp.zeros((npad,), jnp.int32)])
    f_e = _sc_gather(xp, src_pad.reshape(_IDX_ROWS, _CHUNK))
    yt = rsh.T
    radii2d = radii.reshape(1, _N_EDGES)
    w3pt = (W3[:, _W3_PERM] * _W3_SCALE[None, :]).T

    msg_pad = _tc_fused(radii2d, yt, f_e, W0.T, W1.T, W2.T, w3pt)
    dst_pad = jnp.concatenate([dst, jnp.full((npad,), _N_NODES, jnp.int32)])

    zrows = jnp.zeros((_NODES_PER_W, _D_OUT), jnp.float32)
    parts = _sc_scatter(msg_pad, dst_pad.reshape(_IDX_ROWS, _CHUNK), zrows)
    return (parts[0, :_N_NODES] + parts[1, :_N_NODES])[:, _MSG_PERM]


# speedup vs baseline: 1.2142x; 1.0952x over previous
import functools
import math

import jax
import jax.numpy as jnp
import numpy as np
from jax import lax
from jax.experimental import pallas as pl
from jax.experimental.pallas import tpu as pltpu
from jax.experimental.pallas import tpu_sc as plsc

_N_NODES = 10000
_N_EDGES = 160000
_D_IN = 32
_D_OUT = 32
_NUM_BASIS = 10
_H = 100
_R_DIM = 384
_MIN_R, _MAX_R = 0.7, 3.2
_STEP = (_MAX_R - _MIN_R) / (_NUM_BASIS - 1)

_EBLK = 3200
_SUBW = 640
_NSUB = _EBLK // _SUBW
_N_EBLK = _N_EDGES // _EBLK


def _w3j_c(j1, j2, j3, m1, m2, m3):
    if m1 + m2 + m3 != 0 or not (abs(j1 - j2) <= j3 <= j1 + j2):
        return 0.0
    f = math.factorial
    delta = math.sqrt(f(j1 + j2 - j3) * f(j1 - j2 + j3) * f(-j1 + j2 + j3) / f(j1 + j2 + j3 + 1))
    pref = delta * math.sqrt(f(j1 + m1) * f(j1 - m1) * f(j2 + m2) * f(j2 - m2) * f(j3 + m3) * f(j3 - m3))
    kmin = max(0, j2 - j3 - m1, j1 - j3 + m2)
    kmax = min(j1 + j2 - j3, j1 - m1, j2 + m2)
    s = 0.0
    for k in range(kmin, kmax + 1):
        s += (-1.0) ** k / (
            f(k) * f(j1 + j2 - j3 - k) * f(j1 - m1 - k) * f(j2 + m2 - k)
            * f(j3 - j2 + m1 + k) * f(j3 - j1 - m2 + k))
    return ((-1.0) ** (j1 - j2 - m3)) * pref * s


def _u_real(l):
    U = np.zeros((2 * l + 1, 2 * l + 1), dtype=complex)
    s2 = math.sqrt(2.0)
    for m in range(-l, l + 1):
        if m == 0:
            U[l, l] = 1.0
        elif m > 0:
            U[l + m, l - m] = 1.0 / s2
            U[l + m, l + m] = ((-1.0) ** m) / s2
        else:
            a = -m
            U[l + m, l - a] = 1j / s2
            U[l + m, l + a] = -1j * ((-1.0) ** a) / s2
    return U


def _wigner_3j_real(l1, l2, l3):
    C = np.zeros((2 * l1 + 1, 2 * l2 + 1, 2 * l3 + 1), dtype=complex)
    for m1 in range(-l1, l1 + 1):
        for m2 in range(-l2, l2 + 1):
            m3 = -(m1 + m2)
            if -l3 <= m3 <= l3:
                C[m1 + l1, m2 + l2, m3 + l3] = _w3j_c(l1, l2, l3, m1, m2, m3)
    T = np.einsum('am,bn,co,mno->abc', _u_real(l1), _u_real(l2), _u_real(l3), C)
    flat = T.reshape(-1)
    k = int(np.argmax(np.abs(flat)))
    if abs(flat[k]) > 0:
        ph = flat[k] / abs(flat[k])
        T = T * np.conj(ph)
    return np.real(T).astype(np.float64)


_C011 = _wigner_3j_real(0, 1, 1)
_C101 = _wigner_3j_real(1, 0, 1)
_C110 = _wigner_3j_real(1, 1, 0)
_C111 = _wigner_3j_real(1, 1, 1)
_C112 = _wigner_3j_real(1, 1, 2)

_SQ4PI = math.sqrt(4 * math.pi)
_NORM0 = _SQ4PI * math.sqrt(1.0) / math.sqrt(8 * 1 + 8 * 1)
_NORM1 = _SQ4PI * math.sqrt(3.0) / math.sqrt(8 * 1 + 8 * 3)

_DELTA3 = float(_C011[0, 0, 0])
_EPS = float(abs(_C111[0, 1, 2]))
_EPS_TERMS = [[(b, c, float(np.sign(_C111[a, b, c])))
               for b in range(3) for c in range(3)
               if abs(_C111[a, b, c]) > 1e-12] for a in range(3)]
_C112_TERMS = [[[(c, float(_C112[a, b, c])) for c in range(5)
                 if abs(_C112[a, b, c]) > 1e-12] for b in range(3)]
               for a in range(3)]

_INV_SQRT_H = 1.0 / math.sqrt(_H)
_INV_SQRT_B = 1.0 / math.sqrt(_NUM_BASIS)


def _build_w3_perm_scale():
    perm = np.zeros(_R_DIM, dtype=np.int64)
    scale = np.zeros(_R_DIM, dtype=np.float64)
    for v in range(8):
        for u in range(8):
            perm[v * 8 + u] = u * 8 + v
            scale[v * 8 + u] = _NORM0
            perm[64 + v * 8 + u] = 64 + u * 8 + v
            scale[64 + v * 8 + u] = _NORM0 * _DELTA3
            perm[128 + v * 8 + u] = 128 + u * 8 + v
            scale[128 + v * 8 + u] = _NORM1 * _DELTA3
            for fi in range(3):
                perm[192 + (v * 3 + fi) * 8 + u] = 192 + (u * 8 + v) * 3 + fi
                scale[192 + (v * 3 + fi) * 8 + u] = _NORM1 * (
                    _DELTA3 if fi == 0 else (_EPS if fi == 1 else 1.0))
    return perm, (scale * _INV_SQRT_H).astype(np.float32)


_W3_PERM, _W3_SCALE = _build_w3_perm_scale()

_F_PERM = np.concatenate([
    np.arange(8),
    np.array([8 + v * 3 + b for b in range(3) for v in range(8)]),
]).astype(np.int32)

_MSG_PERM = np.concatenate([
    np.arange(8),
    np.array([8 + a * 8 + u for u in range(8) for a in range(3)]),
]).astype(np.int32)


def _silu(z):
    return z * (1.0 / (1.0 + jnp.exp(-z)))


def _tree_sum(xs):
    xs = list(xs)
    while len(xs) > 1:
        nxt = [a + b for a, b in zip(xs[0::2], xs[1::2])]
        if len(xs) % 2:
            nxt.append(xs[-1])
        xs = nxt
    return xs[0]


def _tc_fused_body(radii_ref, yt_ref, ft_ref, w0_ref, w1_ref, w2_ref, w3_ref,
                   out_ref):
    f32 = jnp.float32
    dn = (((1,), (0,)), ((), ()))
    rrow_all = radii_ref[...]
    ft_all = ft_ref[...].T
    yt_all = yt_ref[...]
    centers = (_MIN_R + _STEP * lax.broadcasted_iota(
        jnp.int32, (_NUM_BASIS, _SUBW), 0).astype(f32))

    for sub in range(_NSUB):
        ls = slice(sub * _SUBW, (sub + 1) * _SUBW)
        rrow = rrow_all[:, ls]
        ft = ft_all[:, ls]
        yt = yt_all[:, ls]

        t = (jnp.broadcast_to(rrow, (_NUM_BASIS, _SUBW)) - centers) * (1.0 / _STEP)
        b = jnp.exp(-(t * t))
        h = _silu(lax.dot_general(w0_ref[...], b, dn, preferred_element_type=f32)
                  * _INV_SQRT_B)
        h = _silu(lax.dot_general(w1_ref[...], h, dn, preferred_element_type=f32)
                  * _INV_SQRT_H)
        h = _silu(lax.dot_general(w2_ref[...], h, dn, preferred_element_type=f32)
                  * _INV_SQRT_H)
        rt = lax.dot_general(w3_ref[...], h, dn, preferred_element_type=f32)

        f0 = ft[0:8]
        f1 = [ft[8 + bb * 8: 16 + bb * 8] for bb in range(3)]
        y0 = yt[0:1]
        y1 = [yt[1 + c: 2 + c] for c in range(3)]
        y2 = [yt[4 + c: 5 + c] for c in range(5)]

        pa = f0 * y0
        pb = f1[0] * y1[0] + (f1[1] * y1[1] + f1[2] * y1[2])
        acc0 = _tree_sum(
            [rt[v * 8: v * 8 + 8] * pa[v: v + 1] for v in range(8)]
            + [rt[64 + v * 8: 72 + v * 8] * pb[v: v + 1] for v in range(8)])
        accs = [acc0]

        for a in range(3):
            pc = f0 * y1[a]
            pd0 = f1[a] * y0
            pd1 = _tree_sum([f1[bb] * (s * y1[c])
                             for bb, c, s in _EPS_TERMS[a]])
            pd2 = _tree_sum([
                f1[bb] * _tree_sum([coeff * y2[c]
                                    for c, coeff in _C112_TERMS[a][bb]])
                for bb in range(3)])
            terms = []
            for v in range(8):
                base = 192 + v * 24
                terms += [
                    rt[128 + v * 8: 136 + v * 8] * pc[v: v + 1],
                    rt[base: base + 8] * pd0[v: v + 1],
                    rt[base + 8: base + 16] * pd1[v: v + 1],
                    rt[base + 16: base + 24] * pd2[v: v + 1],
                ]
            accs.append(_tree_sum(terms))
        out_ref[ls, :] = jnp.concatenate(accs, axis=0).T


def _tc_fused(radii2d, yt, fe, w0t, w1t, w2t, w3pt, *, interpret=False):
    const = lambda shape: pl.BlockSpec(shape, lambda i: (0, 0))
    eb = lambda rows: pl.BlockSpec((rows, _EBLK), lambda i: (0, i))
    erow = pl.BlockSpec((_EBLK, _D_IN), lambda i: (i, 0))
    return pl.pallas_call(
        _tc_fused_body,
        grid=(_N_EBLK,),
        in_specs=[
            eb(1),
            eb(9),
            erow,
            const((_H, _NUM_BASIS)),
            const((_H, _H)),
            const((_H, _H)),
            const((_R_DIM, _H)),
        ],
        out_specs=pl.BlockSpec((_EBLK, _D_OUT), lambda i: (i, 0)),
        out_shape=jax.ShapeDtypeStruct((_E_PAD, _D_OUT), jnp.float32),
        interpret=interpret,
    )(radii2d, yt, fe, w0t, w1t, w2t, w3pt)


_NW = 32
_CHUNK = 128
_CHUNKS_PER_W = 40
_E_PAD = _NW * _CHUNKS_PER_W * _CHUNK
_E_PER_W = _E_PAD // _NW
_IDX_ROWS = _E_PAD // _CHUNK
_N_ACC = 10240
_NODES_PER_W = _N_ACC // 16


_SC_PARAMS = pltpu.CompilerParams(use_tc_tiling_on_sc=False)
_NBUF = 4


def _sc_gather(xp, src2d):
    mesh = plsc.VectorSubcoreMesh(core_axis_name="c", subcore_axis_name="s")

    @functools.partial(
        pl.kernel,
        out_type=jax.ShapeDtypeStruct((_E_PAD, _D_IN), jnp.float32),
        mesh=mesh,
        scratch_types=[
            pltpu.VMEM((_CHUNKS_PER_W, _CHUNK), jnp.int32),
            pltpu.VMEM((_NBUF, _CHUNK, _D_IN), jnp.float32),
        ] + [pltpu.SemaphoreType.DMA] * _NBUF,
        compiler_params=_SC_PARAMS,
    )
    def k(x_hbm, idx_hbm, out_hbm, idx_v, bufs, *sems):
        wid = lax.axis_index("c") * 16 + lax.axis_index("s")
        pltpu.sync_copy(
            idx_hbm.at[pl.ds(wid * _CHUNKS_PER_W, _CHUNKS_PER_W)], idx_v)

        def body(t, carry):
            hs = []
            for p in range(_NBUF):
                j = t * _NBUF + p
                hs.append(pltpu.async_copy(
                    x_hbm.at[idx_v.at[j]], bufs.at[p], sems[p]))
            for p in range(_NBUF):
                j = t * _NBUF + p
                hs[p].wait()
                pltpu.sync_copy(
                    bufs.at[p],
                    out_hbm.at[pl.ds(wid * _E_PER_W + j * _CHUNK, _CHUNK)])
            return carry

        lax.fori_loop(0, _CHUNKS_PER_W // _NBUF, body, 0)

    return k(xp, src2d)


def _sc_scatter(msg, dst2d, zrows):
    mesh = plsc.VectorSubcoreMesh(core_axis_name="c", subcore_axis_name="s")

    @functools.partial(
        pl.kernel,
        out_type=jax.ShapeDtypeStruct((2, _N_ACC, _D_OUT), jnp.float32),
        mesh=mesh,
        scratch_types=[
            pltpu.VMEM((_CHUNKS_PER_W, _CHUNK), jnp.int32),
            pltpu.VMEM((_NBUF, _CHUNK, _D_OUT), jnp.float32),
            pltpu.VMEM_SHARED((_N_ACC, _D_OUT), jnp.float32),
        ] + [pltpu.SemaphoreType.DMA] * _NBUF,
        compiler_params=_SC_PARAMS,
    )
    def k(msg_hbm, idx_hbm, z_hbm, out_hbm, idx_v, bufs, acc_sh, *sems):
        c = lax.axis_index("c")
        s = lax.axis_index("s")
        wid = c * 16 + s
        pltpu.sync_copy(z_hbm, acc_sh.at[pl.ds(s * _NODES_PER_W, _NODES_PER_W)])
        pltpu.sync_copy(
            idx_hbm.at[pl.ds(wid * _CHUNKS_PER_W, _CHUNKS_PER_W)], idx_v)
        plsc.subcore_barrier()

        def body(t, carry):
            hs = []
            for p in range(_NBUF):
                j = t * _NBUF + p
                hs.append(pltpu.async_copy(
                    msg_hbm.at[pl.ds(wid * _E_PER_W + j * _CHUNK, _CHUNK)],
                    bufs.at[p], sems[p]))
            for p in range(_NBUF):
                j = t * _NBUF + p
                hs[p].wait()
                pltpu.sync_copy(bufs.at[p], acc_sh.at[idx_v.at[j]], add=True)
            return carry

        lax.fori_loop(0, _CHUNKS_PER_W // _NBUF, body, 0)
        plsc.subcore_barrier()
        pltpu.sync_copy(
            acc_sh.at[pl.ds(s * _NODES_PER_W, _NODES_PER_W)],
            out_hbm.at[c, pl.ds(s * _NODES_PER_W, _NODES_PER_W)])

    return k(msg, dst2d, zrows)


@jax.jit
def kernel(x, edge_index, radii, rsh, W0, W1, W2, W3):
    src = edge_index[0].astype(jnp.int32)
    dst = edge_index[1].astype(jnp.int32)

    npad = _E_PAD - _N_EDGES
    xp = x[:, _F_PERM]
    src_pad = jnp.concatenate([src, jnp.zeros((npad,), jnp.int32)])
    f_e = _sc_gather(xp, src_pad.reshape(_IDX_ROWS, _CHUNK))
    yt = rsh.T
    radii2d = radii.reshape(1, _N_EDGES)
    w3pt = (W3[:, _W3_PERM] * _W3_SCALE[None, :]).T

    msg_pad = _tc_fused(radii2d, yt, f_e, W0.T, W1.T, W2.T, w3pt)
    dst_pad = jnp.concatenate([dst, jnp.full((npad,), _N_NODES, jnp.int32)])

    zrows = jnp.zeros((_NODES_PER_W, _D_OUT), jnp.float32)
    parts = _sc_scatter(msg_pad, dst_pad.reshape(_IDX_ROWS, _CHUNK), zrows)
    return (parts[0, :_N_NODES] + parts[1, :_N_NODES])[:, _MSG_PERM]


# speedup vs baseline: 1.2344x; 1.0166x over previous
import functools
import math

import jax
import jax.numpy as jnp
import numpy as np
from jax import lax
from jax.experimental import pallas as pl
from jax.experimental.pallas import tpu as pltpu
from jax.experimental.pallas import tpu_sc as plsc

_N_NODES = 10000
_N_EDGES = 160000
_D_IN = 32
_D_OUT = 32
_NUM_BASIS = 10
_H = 100
_R_DIM = 384
_MIN_R, _MAX_R = 0.7, 3.2
_STEP = (_MAX_R - _MIN_R) / (_NUM_BASIS - 1)

_EBLK = 6400
_SUBW = 640
_NSUB = _EBLK // _SUBW
_N_EBLK = _N_EDGES // _EBLK


def _w3j_c(j1, j2, j3, m1, m2, m3):
    if m1 + m2 + m3 != 0 or not (abs(j1 - j2) <= j3 <= j1 + j2):
        return 0.0
    f = math.factorial
    delta = math.sqrt(f(j1 + j2 - j3) * f(j1 - j2 + j3) * f(-j1 + j2 + j3) / f(j1 + j2 + j3 + 1))
    pref = delta * math.sqrt(f(j1 + m1) * f(j1 - m1) * f(j2 + m2) * f(j2 - m2) * f(j3 + m3) * f(j3 - m3))
    kmin = max(0, j2 - j3 - m1, j1 - j3 + m2)
    kmax = min(j1 + j2 - j3, j1 - m1, j2 + m2)
    s = 0.0
    for k in range(kmin, kmax + 1):
        s += (-1.0) ** k / (
            f(k) * f(j1 + j2 - j3 - k) * f(j1 - m1 - k) * f(j2 + m2 - k)
            * f(j3 - j2 + m1 + k) * f(j3 - j1 - m2 + k))
    return ((-1.0) ** (j1 - j2 - m3)) * pref * s


def _u_real(l):
    U = np.zeros((2 * l + 1, 2 * l + 1), dtype=complex)
    s2 = math.sqrt(2.0)
    for m in range(-l, l + 1):
        if m == 0:
            U[l, l] = 1.0
        elif m > 0:
            U[l + m, l - m] = 1.0 / s2
            U[l + m, l + m] = ((-1.0) ** m) / s2
        else:
            a = -m
            U[l + m, l - a] = 1j / s2
            U[l + m, l + a] = -1j * ((-1.0) ** a) / s2
    return U


def _wigner_3j_real(l1, l2, l3):
    C = np.zeros((2 * l1 + 1, 2 * l2 + 1, 2 * l3 + 1), dtype=complex)
    for m1 in range(-l1, l1 + 1):
        for m2 in range(-l2, l2 + 1):
            m3 = -(m1 + m2)
            if -l3 <= m3 <= l3:
                C[m1 + l1, m2 + l2, m3 + l3] = _w3j_c(l1, l2, l3, m1, m2, m3)
    T = np.einsum('am,bn,co,mno->abc', _u_real(l1), _u_real(l2), _u_real(l3), C)
    flat = T.reshape(-1)
    k = int(np.argmax(np.abs(flat)))
    if abs(flat[k]) > 0:
        ph = flat[k] / abs(flat[k])
        T = T * np.conj(ph)
    return np.real(T).astype(np.float64)


_C011 = _wigner_3j_real(0, 1, 1)
_C101 = _wigner_3j_real(1, 0, 1)
_C110 = _wigner_3j_real(1, 1, 0)
_C111 = _wigner_3j_real(1, 1, 1)
_C112 = _wigner_3j_real(1, 1, 2)

_SQ4PI = math.sqrt(4 * math.pi)
_NORM0 = _SQ4PI * math.sqrt(1.0) / math.sqrt(8 * 1 + 8 * 1)
_NORM1 = _SQ4PI * math.sqrt(3.0) / math.sqrt(8 * 1 + 8 * 3)

_DELTA3 = float(_C011[0, 0, 0])
_EPS = float(abs(_C111[0, 1, 2]))
_EPS_TERMS = [[(b, c, float(np.sign(_C111[a, b, c])))
               for b in range(3) for c in range(3)
               if abs(_C111[a, b, c]) > 1e-12] for a in range(3)]
_C112_TERMS = [[[(c, float(_C112[a, b, c])) for c in range(5)
                 if abs(_C112[a, b, c]) > 1e-12] for b in range(3)]
               for a in range(3)]

_INV_SQRT_H = 1.0 / math.sqrt(_H)
_INV_SQRT_B = 1.0 / math.sqrt(_NUM_BASIS)


def _build_w3_perm_scale():
    perm = np.zeros(_R_DIM, dtype=np.int64)
    scale = np.zeros(_R_DIM, dtype=np.float64)
    for v in range(8):
        for u in range(8):
            perm[v * 8 + u] = u * 8 + v
            scale[v * 8 + u] = _NORM0
            perm[64 + v * 8 + u] = 64 + u * 8 + v
            scale[64 + v * 8 + u] = _NORM0 * _DELTA3
            perm[128 + v * 8 + u] = 128 + u * 8 + v
            scale[128 + v * 8 + u] = _NORM1 * _DELTA3
            for fi in range(3):
                perm[192 + (v * 3 + fi) * 8 + u] = 192 + (u * 8 + v) * 3 + fi
                scale[192 + (v * 3 + fi) * 8 + u] = _NORM1 * (
                    _DELTA3 if fi == 0 else (_EPS if fi == 1 else 1.0))
    return perm, (scale * _INV_SQRT_H).astype(np.float32)


_W3_PERM, _W3_SCALE = _build_w3_perm_scale()

_F_PERM = np.concatenate([
    np.arange(8),
    np.array([8 + v * 3 + b for b in range(3) for v in range(8)]),
]).astype(np.int32)

_MSG_PERM = np.concatenate([
    np.arange(8),
    np.array([8 + a * 8 + u for u in range(8) for a in range(3)]),
]).astype(np.int32)


def _silu(z):
    return z * (1.0 / (1.0 + jnp.exp(-z)))


def _tree_sum(xs):
    xs = list(xs)
    while len(xs) > 1:
        nxt = [a + b for a, b in zip(xs[0::2], xs[1::2])]
        if len(xs) % 2:
            nxt.append(xs[-1])
        xs = nxt
    return xs[0]


def _tc_fused_body(radii_ref, yt_ref, ft_ref, w0_ref, w1_ref, w2_ref, w3_ref,
                   out_ref):
    f32 = jnp.float32
    dn = (((1,), (0,)), ((), ()))
    rrow_all = radii_ref[...]
    ft_all = ft_ref[...].T
    yt_all = yt_ref[...]
    centers = (_MIN_R + _STEP * lax.broadcasted_iota(
        jnp.int32, (_NUM_BASIS, _SUBW), 0).astype(f32))

    for sub in range(_NSUB):
        ls = slice(sub * _SUBW, (sub + 1) * _SUBW)
        rrow = rrow_all[:, ls]
        ft = ft_all[:, ls]
        yt = yt_all[:, ls]

        t = (jnp.broadcast_to(rrow, (_NUM_BASIS, _SUBW)) - centers) * (1.0 / _STEP)
        b = jnp.exp(-(t * t))
        h = _silu(lax.dot_general(w0_ref[...], b, dn, preferred_element_type=f32)
                  * _INV_SQRT_B)
        h = _silu(lax.dot_general(w1_ref[...], h, dn, preferred_element_type=f32)
                  * _INV_SQRT_H)
        h = _silu(lax.dot_general(w2_ref[...], h, dn, preferred_element_type=f32)
                  * _INV_SQRT_H)
        rt = lax.dot_general(w3_ref[...], h, dn, preferred_element_type=f32)

        f0 = ft[0:8]
        f1 = [ft[8 + bb * 8: 16 + bb * 8] for bb in range(3)]
        y0 = yt[0:1]
        y1 = [yt[1 + c: 2 + c] for c in range(3)]
        y2 = [yt[4 + c: 5 + c] for c in range(5)]

        pa = f0 * y0
        pb = f1[0] * y1[0] + (f1[1] * y1[1] + f1[2] * y1[2])
        acc0 = _tree_sum(
            [rt[v * 8: v * 8 + 8] * pa[v: v + 1] for v in range(8)]
            + [rt[64 + v * 8: 72 + v * 8] * pb[v: v + 1] for v in range(8)])
        accs = [acc0]

        for a in range(3):
            pc = f0 * y1[a]
            pd0 = f1[a] * y0
            pd1 = _tree_sum([f1[bb] * (s * y1[c])
                             for bb, c, s in _EPS_TERMS[a]])
            pd2 = _tree_sum([
                f1[bb] * _tree_sum([coeff * y2[c]
                                    for c, coeff in _C112_TERMS[a][bb]])
                for bb in range(3)])
            terms = []
            for v in range(8):
                base = 192 + v * 24
                terms += [
                    rt[128 + v * 8: 136 + v * 8] * pc[v: v + 1],
                    rt[base: base + 8] * pd0[v: v + 1],
                    rt[base + 8: base + 16] * pd1[v: v + 1],
                    rt[base + 16: base + 24] * pd2[v: v + 1],
                ]
            accs.append(_tree_sum(terms))
        out_ref[ls, :] = jnp.concatenate(accs, axis=0).T


def _tc_fused(radii2d, yt, fe, w0t, w1t, w2t, w3pt, *, interpret=False):
    const = lambda shape: pl.BlockSpec(shape, lambda i: (0, 0))
    eb = lambda rows: pl.BlockSpec((rows, _EBLK), lambda i: (0, i))
    erow = pl.BlockSpec((_EBLK, _D_IN), lambda i: (i, 0))
    return pl.pallas_call(
        _tc_fused_body,
        grid=(_N_EBLK,),
        in_specs=[
            eb(1),
            eb(9),
            erow,
            const((_H, _NUM_BASIS)),
            const((_H, _H)),
            const((_H, _H)),
            const((_R_DIM, _H)),
        ],
        out_specs=pl.BlockSpec((_EBLK, _D_OUT), lambda i: (i, 0)),
        out_shape=jax.ShapeDtypeStruct((_E_PAD, _D_OUT), jnp.float32),
        interpret=interpret,
    )(radii2d, yt, fe, w0t, w1t, w2t, w3pt)


_NW = 32
_CHUNK = 128
_CHUNKS_PER_W = 40
_E_PAD = _NW * _CHUNKS_PER_W * _CHUNK
_E_PER_W = _E_PAD // _NW
_IDX_ROWS = _E_PAD // _CHUNK
_N_ACC = 10240
_NODES_PER_W = _N_ACC // 16


_SC_PARAMS = pltpu.CompilerParams(use_tc_tiling_on_sc=False)
_NBUF = 4


def _sc_gather(xp, src2d):
    mesh = plsc.VectorSubcoreMesh(core_axis_name="c", subcore_axis_name="s")

    @functools.partial(
        pl.kernel,
        out_type=jax.ShapeDtypeStruct((_E_PAD, _D_IN), jnp.float32),
        mesh=mesh,
        scratch_types=[
            pltpu.VMEM((_CHUNKS_PER_W, _CHUNK), jnp.int32),
            pltpu.VMEM((_NBUF, _CHUNK, _D_IN), jnp.float32),
        ] + [pltpu.SemaphoreType.DMA] * _NBUF,
        compiler_params=_SC_PARAMS,
    )
    def k(x_hbm, idx_hbm, out_hbm, idx_v, bufs, *sems):
        wid = lax.axis_index("c") * 16 + lax.axis_index("s")
        pltpu.sync_copy(
            idx_hbm.at[pl.ds(wid * _CHUNKS_PER_W, _CHUNKS_PER_W)], idx_v)

        def body(t, carry):
            hs = []
            for p in range(_NBUF):
                j = t * _NBUF + p
                hs.append(pltpu.async_copy(
                    x_hbm.at[idx_v.at[j]], bufs.at[p], sems[p]))
            for p in range(_NBUF):
                j = t * _NBUF + p
                hs[p].wait()
                pltpu.sync_copy(
                    bufs.at[p],
                    out_hbm.at[pl.ds(wid * _E_PER_W + j * _CHUNK, _CHUNK)])
            return carry

        lax.fori_loop(0, _CHUNKS_PER_W // _NBUF, body, 0)

    return k(xp, src2d)


def _sc_scatter(msg, dst2d, zrows):
    mesh = plsc.VectorSubcoreMesh(core_axis_name="c", subcore_axis_name="s")

    @functools.partial(
        pl.kernel,
        out_type=jax.ShapeDtypeStruct((2, _N_ACC, _D_OUT), jnp.float32),
        mesh=mesh,
        scratch_types=[
            pltpu.VMEM((_CHUNKS_PER_W, _CHUNK), jnp.int32),
            pltpu.VMEM((_NBUF, _CHUNK, _D_OUT), jnp.float32),
            pltpu.VMEM_SHARED((_N_ACC, _D_OUT), jnp.float32),
        ] + [pltpu.SemaphoreType.DMA] * _NBUF,
        compiler_params=_SC_PARAMS,
    )
    def k(msg_hbm, idx_hbm, z_hbm, out_hbm, idx_v, bufs, acc_sh, *sems):
        c = lax.axis_index("c")
        s = lax.axis_index("s")
        wid = c * 16 + s
        pltpu.sync_copy(z_hbm, acc_sh.at[pl.ds(s * _NODES_PER_W, _NODES_PER_W)])
        pltpu.sync_copy(
            idx_hbm.at[pl.ds(wid * _CHUNKS_PER_W, _CHUNKS_PER_W)], idx_v)
        plsc.subcore_barrier()

        def body(t, carry):
            hs = []
            for p in range(_NBUF):
                j = t * _NBUF + p
                hs.append(pltpu.async_copy(
                    msg_hbm.at[pl.ds(wid * _E_PER_W + j * _CHUNK, _CHUNK)],
                    bufs.at[p], sems[p]))
            for p in range(_NBUF):
                j = t * _NBUF + p
                hs[p].wait()
                pltpu.sync_copy(bufs.at[p], acc_sh.at[idx_v.at[j]], add=True)
            return carry

        lax.fori_loop(0, _CHUNKS_PER_W // _NBUF, body, 0)
        plsc.subcore_barrier()
        pltpu.sync_copy(
            acc_sh.at[pl.ds(s * _NODES_PER_W, _NODES_PER_W)],
            out_hbm.at[c, pl.ds(s * _NODES_PER_W, _NODES_PER_W)])

    return k(msg, dst2d, zrows)


@jax.jit
def kernel(x, edge_index, radii, rsh, W0, W1, W2, W3):
    src = edge_index[0].astype(jnp.int32)
    dst = edge_index[1].astype(jnp.int32)

    npad = _E_PAD - _N_EDGES
    xp = x[:, _F_PERM]
    src_pad = jnp.concatenate([src, jnp.zeros((npad,), jnp.int32)])
    f_e = _sc_gather(xp, src_pad.reshape(_IDX_ROWS, _CHUNK))
    yt = rsh.T
    radii2d = radii.reshape(1, _N_EDGES)
    w3pt = (W3[:, _W3_PERM] * _W3_SCALE[None, :]).T

    msg_pad = _tc_fused(radii2d, yt, f_e, W0.T, W1.T, W2.T, w3pt)
    dst_pad = jnp.concatenate([dst, jnp.full((npad,), _N_NODES, jnp.int32)])

    zrows = jnp.zeros((_NODES_PER_W, _D_OUT), jnp.float32)
    parts = _sc_scatter(msg_pad, dst_pad.reshape(_IDX_ROWS, _CHUNK), zrows)
    return (parts[0, :_N_NODES] + parts[1, :_N_NODES])[:, _MSG_PERM]


# speedup vs baseline: 1.3732x; 1.1125x over previous
import functools
import math

import jax
import jax.numpy as jnp
import numpy as np
from jax import lax
from jax.experimental import pallas as pl
from jax.experimental.pallas import tpu as pltpu
from jax.experimental.pallas import tpu_sc as plsc

_N_NODES = 10000
_N_EDGES = 160000
_D_IN = 32
_D_OUT = 32
_NUM_BASIS = 10
_H = 100
_R_DIM = 384
_MIN_R, _MAX_R = 0.7, 3.2
_STEP = (_MAX_R - _MIN_R) / (_NUM_BASIS - 1)

_EBLK = 6400
_SUBW = 1280
_NSUB = _EBLK // _SUBW
_N_EBLK = _N_EDGES // _EBLK


def _w3j_c(j1, j2, j3, m1, m2, m3):
    if m1 + m2 + m3 != 0 or not (abs(j1 - j2) <= j3 <= j1 + j2):
        return 0.0
    f = math.factorial
    delta = math.sqrt(f(j1 + j2 - j3) * f(j1 - j2 + j3) * f(-j1 + j2 + j3) / f(j1 + j2 + j3 + 1))
    pref = delta * math.sqrt(f(j1 + m1) * f(j1 - m1) * f(j2 + m2) * f(j2 - m2) * f(j3 + m3) * f(j3 - m3))
    kmin = max(0, j2 - j3 - m1, j1 - j3 + m2)
    kmax = min(j1 + j2 - j3, j1 - m1, j2 + m2)
    s = 0.0
    for k in range(kmin, kmax + 1):
        s += (-1.0) ** k / (
            f(k) * f(j1 + j2 - j3 - k) * f(j1 - m1 - k) * f(j2 + m2 - k)
            * f(j3 - j2 + m1 + k) * f(j3 - j1 - m2 + k))
    return ((-1.0) ** (j1 - j2 - m3)) * pref * s


def _u_real(l):
    U = np.zeros((2 * l + 1, 2 * l + 1), dtype=complex)
    s2 = math.sqrt(2.0)
    for m in range(-l, l + 1):
        if m == 0:
            U[l, l] = 1.0
        elif m > 0:
            U[l + m, l - m] = 1.0 / s2
            U[l + m, l + m] = ((-1.0) ** m) / s2
        else:
            a = -m
            U[l + m, l - a] = 1j / s2
            U[l + m, l + a] = -1j * ((-1.0) ** a) / s2
    return U


def _wigner_3j_real(l1, l2, l3):
    C = np.zeros((2 * l1 + 1, 2 * l2 + 1, 2 * l3 + 1), dtype=complex)
    for m1 in range(-l1, l1 + 1):
        for m2 in range(-l2, l2 + 1):
            m3 = -(m1 + m2)
            if -l3 <= m3 <= l3:
                C[m1 + l1, m2 + l2, m3 + l3] = _w3j_c(l1, l2, l3, m1, m2, m3)
    T = np.einsum('am,bn,co,mno->abc', _u_real(l1), _u_real(l2), _u_real(l3), C)
    flat = T.reshape(-1)
    k = int(np.argmax(np.abs(flat)))
    if abs(flat[k]) > 0:
        ph = flat[k] / abs(flat[k])
        T = T * np.conj(ph)
    return np.real(T).astype(np.float64)


_C011 = _wigner_3j_real(0, 1, 1)
_C101 = _wigner_3j_real(1, 0, 1)
_C110 = _wigner_3j_real(1, 1, 0)
_C111 = _wigner_3j_real(1, 1, 1)
_C112 = _wigner_3j_real(1, 1, 2)

_SQ4PI = math.sqrt(4 * math.pi)
_NORM0 = _SQ4PI * math.sqrt(1.0) / math.sqrt(8 * 1 + 8 * 1)
_NORM1 = _SQ4PI * math.sqrt(3.0) / math.sqrt(8 * 1 + 8 * 3)

_DELTA3 = float(_C011[0, 0, 0])
_EPS = float(abs(_C111[0, 1, 2]))
_EPS_TERMS = [[(b, c, float(np.sign(_C111[a, b, c])))
               for b in range(3) for c in range(3)
               if abs(_C111[a, b, c]) > 1e-12] for a in range(3)]
_C112_TERMS = [[[(c, float(_C112[a, b, c])) for c in range(5)
                 if abs(_C112[a, b, c]) > 1e-12] for b in range(3)]
               for a in range(3)]

_INV_SQRT_H = 1.0 / math.sqrt(_H)
_INV_SQRT_B = 1.0 / math.sqrt(_NUM_BASIS)


def _build_w3_perm_scale():
    perm = np.zeros(_R_DIM, dtype=np.int64)
    scale = np.zeros(_R_DIM, dtype=np.float64)
    for v in range(8):
        for u in range(8):
            perm[v * 8 + u] = u * 8 + v
            scale[v * 8 + u] = _NORM0
            perm[64 + v * 8 + u] = 64 + u * 8 + v
            scale[64 + v * 8 + u] = _NORM0 * _DELTA3
            perm[128 + v * 8 + u] = 128 + u * 8 + v
            scale[128 + v * 8 + u] = _NORM1 * _DELTA3
            for fi in range(3):
                perm[192 + (v * 3 + fi) * 8 + u] = 192 + (u * 8 + v) * 3 + fi
                scale[192 + (v * 3 + fi) * 8 + u] = _NORM1 * (
                    _DELTA3 if fi == 0 else (_EPS if fi == 1 else 1.0))
    return perm, (scale * _INV_SQRT_H).astype(np.float32)


_W3_PERM, _W3_SCALE = _build_w3_perm_scale()

_F_PERM = np.concatenate([
    np.arange(8),
    np.array([8 + v * 3 + b for b in range(3) for v in range(8)]),
]).astype(np.int32)

_MSG_PERM = np.concatenate([
    np.arange(8),
    np.array([8 + a * 8 + u for u in range(8) for a in range(3)]),
]).astype(np.int32)


def _silu(z):
    return z * (1.0 / (1.0 + jnp.exp(-z)))


def _tree_sum(xs):
    xs = list(xs)
    while len(xs) > 1:
        nxt = [a + b for a, b in zip(xs[0::2], xs[1::2])]
        if len(xs) % 2:
            nxt.append(xs[-1])
        xs = nxt
    return xs[0]


def _tc_fused_body(radii_ref, yt_ref, ft_ref, w0_ref, w1_ref, w2_ref, w3_ref,
                   out_ref):
    f32 = jnp.float32
    dn = (((1,), (0,)), ((), ()))
    rrow_all = radii_ref[...]
    ft_all = ft_ref[...].T
    yt_all = yt_ref[...]
    centers = (_MIN_R + _STEP * lax.broadcasted_iota(
        jnp.int32, (_NUM_BASIS, _SUBW), 0).astype(f32))

    for sub in range(_NSUB):
        ls = slice(sub * _SUBW, (sub + 1) * _SUBW)
        rrow = rrow_all[:, ls]
        ft = ft_all[:, ls]
        yt = yt_all[:, ls]

        t = (jnp.broadcast_to(rrow, (_NUM_BASIS, _SUBW)) - centers) * (1.0 / _STEP)
        b = jnp.exp(-(t * t))
        h = _silu(lax.dot_general(w0_ref[...], b, dn, preferred_element_type=f32)
                  * _INV_SQRT_B)
        h = _silu(lax.dot_general(w1_ref[...], h, dn, preferred_element_type=f32)
                  * _INV_SQRT_H)
        h = _silu(lax.dot_general(w2_ref[...], h, dn, preferred_element_type=f32)
                  * _INV_SQRT_H)
        rt = lax.dot_general(w3_ref[...], h, dn, preferred_element_type=f32)

        f0 = ft[0:8]
        f1 = [ft[8 + bb * 8: 16 + bb * 8] for bb in range(3)]
        y0 = yt[0:1]
        y1 = [yt[1 + c: 2 + c] for c in range(3)]
        y2 = [yt[4 + c: 5 + c] for c in range(5)]

        pa = f0 * y0
        pb = f1[0] * y1[0] + (f1[1] * y1[1] + f1[2] * y1[2])
        acc0 = _tree_sum(
            [rt[v * 8: v * 8 + 8] * pa[v: v + 1] for v in range(8)]
            + [rt[64 + v * 8: 72 + v * 8] * pb[v: v + 1] for v in range(8)])
        accs = [acc0]

        for a in range(3):
            pc = f0 * y1[a]
            pd0 = f1[a] * y0
            pd1 = _tree_sum([f1[bb] * (s * y1[c])
                             for bb, c, s in _EPS_TERMS[a]])
            pd2 = _tree_sum([
                f1[bb] * _tree_sum([coeff * y2[c]
                                    for c, coeff in _C112_TERMS[a][bb]])
                for bb in range(3)])
            terms = []
            for v in range(8):
                base = 192 + v * 24
                terms += [
                    rt[128 + v * 8: 136 + v * 8] * pc[v: v + 1],
                    rt[base: base + 8] * pd0[v: v + 1],
                    rt[base + 8: base + 16] * pd1[v: v + 1],
                    rt[base + 16: base + 24] * pd2[v: v + 1],
                ]
            accs.append(_tree_sum(terms))
        out_ref[ls, :] = jnp.concatenate(accs, axis=0).T


def _tc_fused(radii2d, yt, fe, w0t, w1t, w2t, w3pt, *, interpret=False):
    const = lambda shape: pl.BlockSpec(shape, lambda i: (0, 0))
    eb = lambda rows: pl.BlockSpec((rows, _EBLK), lambda i: (0, i))
    erow = pl.BlockSpec((_EBLK, _D_IN), lambda i: (i, 0))
    return pl.pallas_call(
        _tc_fused_body,
        grid=(_N_EBLK,),
        in_specs=[
            eb(1),
            eb(9),
            erow,
            const((_H, _NUM_BASIS)),
            const((_H, _H)),
            const((_H, _H)),
            const((_R_DIM, _H)),
        ],
        out_specs=pl.BlockSpec((_EBLK, _D_OUT), lambda i: (i, 0)),
        out_shape=jax.ShapeDtypeStruct((_E_PAD, _D_OUT), jnp.float32),
        interpret=interpret,
    )(radii2d, yt, fe, w0t, w1t, w2t, w3pt)


_NW = 32
_CHUNK = 128
_CHUNKS_PER_W = 40
_E_PAD = _NW * _CHUNKS_PER_W * _CHUNK
_E_PER_W = _E_PAD // _NW
_IDX_ROWS = _E_PAD // _CHUNK
_N_ACC = 10240
_NODES_PER_W = _N_ACC // 16


_SC_PARAMS = pltpu.CompilerParams(use_tc_tiling_on_sc=False)
_NBUF = 4


def _sc_gather(xp, src2d):
    mesh = plsc.VectorSubcoreMesh(core_axis_name="c", subcore_axis_name="s")

    @functools.partial(
        pl.kernel,
        out_type=jax.ShapeDtypeStruct((_E_PAD, _D_IN), jnp.float32),
        mesh=mesh,
        scratch_types=[
            pltpu.VMEM((_CHUNKS_PER_W, _CHUNK), jnp.int32),
            pltpu.VMEM((_NBUF, _CHUNK, _D_IN), jnp.float32),
        ] + [pltpu.SemaphoreType.DMA] * _NBUF,
        compiler_params=_SC_PARAMS,
    )
    def k(x_hbm, idx_hbm, out_hbm, idx_v, bufs, *sems):
        wid = lax.axis_index("c") * 16 + lax.axis_index("s")
        pltpu.sync_copy(
            idx_hbm.at[pl.ds(wid * _CHUNKS_PER_W, _CHUNKS_PER_W)], idx_v)

        def body(t, carry):
            hs = []
            for p in range(_NBUF):
                j = t * _NBUF + p
                hs.append(pltpu.async_copy(
                    x_hbm.at[idx_v.at[j]], bufs.at[p], sems[p]))
            for p in range(_NBUF):
                j = t * _NBUF + p
                hs[p].wait()
                pltpu.sync_copy(
                    bufs.at[p],
                    out_hbm.at[pl.ds(wid * _E_PER_W + j * _CHUNK, _CHUNK)])
            return carry

        lax.fori_loop(0, _CHUNKS_PER_W // _NBUF, body, 0)

    return k(xp, src2d)


def _sc_scatter(msg, dst2d, zrows):
    mesh = plsc.VectorSubcoreMesh(core_axis_name="c", subcore_axis_name="s")

    @functools.partial(
        pl.kernel,
        out_type=jax.ShapeDtypeStruct((2, _N_ACC, _D_OUT), jnp.float32),
        mesh=mesh,
        scratch_types=[
            pltpu.VMEM((_CHUNKS_PER_W, _CHUNK), jnp.int32),
            pltpu.VMEM((_NBUF, _CHUNK, _D_OUT), jnp.float32),
            pltpu.VMEM_SHARED((_N_ACC, _D_OUT), jnp.float32),
        ] + [pltpu.SemaphoreType.DMA] * _NBUF,
        compiler_params=_SC_PARAMS,
    )
    def k(msg_hbm, idx_hbm, z_hbm, out_hbm, idx_v, bufs, acc_sh, *sems):
        c = lax.axis_index("c")
        s = lax.axis_index("s")
        wid = c * 16 + s
        pltpu.sync_copy(z_hbm, acc_sh.at[pl.ds(s * _NODES_PER_W, _NODES_PER_W)])
        pltpu.sync_copy(
            idx_hbm.at[pl.ds(wid * _CHUNKS_PER_W, _CHUNKS_PER_W)], idx_v)
        plsc.subcore_barrier()

        def body(t, carry):
            hs = []
            for p in range(_NBUF):
                j = t * _NBUF + p
                hs.append(pltpu.async_copy(
                    msg_hbm.at[pl.ds(wid * _E_PER_W + j * _CHUNK, _CHUNK)],
                    bufs.at[p], sems[p]))
            for p in range(_NBUF):
                j = t * _NBUF + p
                hs[p].wait()
                pltpu.sync_copy(bufs.at[p], acc_sh.at[idx_v.at[j]], add=True)
            return carry

        lax.fori_loop(0, _CHUNKS_PER_W // _NBUF, body, 0)
        plsc.subcore_barrier()
        pltpu.sync_copy(
            acc_sh.at[pl.ds(s * _NODES_PER_W, _NODES_PER_W)],
            out_hbm.at[c, pl.ds(s * _NODES_PER_W, _NODES_PER_W)])

    return k(msg, dst2d, zrows)


@jax.jit
def kernel(x, edge_index, radii, rsh, W0, W1, W2, W3):
    src = edge_index[0].astype(jnp.int32)
    dst = edge_index[1].astype(jnp.int32)

    npad = _E_PAD - _N_EDGES
    xp = x[:, _F_PERM]
    src_pad = jnp.concatenate([src, jnp.zeros((npad,), jnp.int32)])
    f_e = _sc_gather(xp, src_pad.reshape(_IDX_ROWS, _CHUNK))
    yt = rsh.T
    radii2d = radii.reshape(1, _N_EDGES)
    w3pt = (W3[:, _W3_PERM] * _W3_SCALE[None, :]).T

    msg_pad = _tc_fused(radii2d, yt, f_e, W0.T, W1.T, W2.T, w3pt)
    dst_pad = jnp.concatenate([dst, jnp.full((npad,), _N_NODES, jnp.int32)])

    zrows = jnp.zeros((_NODES_PER_W, _D_OUT), jnp.float32)
    parts = _sc_scatter(msg_pad, dst_pad.reshape(_IDX_ROWS, _CHUNK), zrows)
    return (parts[0, :_N_NODES] + parts[1, :_N_NODES])[:, _MSG_PERM]


# speedup vs baseline: 1.4105x; 1.0271x over previous
import functools
import math

import jax
import jax.numpy as jnp
import numpy as np
from jax import lax
from jax.experimental import pallas as pl
from jax.experimental.pallas import tpu as pltpu
from jax.experimental.pallas import tpu_sc as plsc

_N_NODES = 10000
_N_EDGES = 160000
_D_IN = 32
_D_OUT = 32
_NUM_BASIS = 10
_H = 100
_R_DIM = 384
_MIN_R, _MAX_R = 0.7, 3.2
_STEP = (_MAX_R - _MIN_R) / (_NUM_BASIS - 1)

_EBLK = 6400
_SUBW = 3200
_NSUB = _EBLK // _SUBW
_N_EBLK = _N_EDGES // _EBLK


def _w3j_c(j1, j2, j3, m1, m2, m3):
    if m1 + m2 + m3 != 0 or not (abs(j1 - j2) <= j3 <= j1 + j2):
        return 0.0
    f = math.factorial
    delta = math.sqrt(f(j1 + j2 - j3) * f(j1 - j2 + j3) * f(-j1 + j2 + j3) / f(j1 + j2 + j3 + 1))
    pref = delta * math.sqrt(f(j1 + m1) * f(j1 - m1) * f(j2 + m2) * f(j2 - m2) * f(j3 + m3) * f(j3 - m3))
    kmin = max(0, j2 - j3 - m1, j1 - j3 + m2)
    kmax = min(j1 + j2 - j3, j1 - m1, j2 + m2)
    s = 0.0
    for k in range(kmin, kmax + 1):
        s += (-1.0) ** k / (
            f(k) * f(j1 + j2 - j3 - k) * f(j1 - m1 - k) * f(j2 + m2 - k)
            * f(j3 - j2 + m1 + k) * f(j3 - j1 - m2 + k))
    return ((-1.0) ** (j1 - j2 - m3)) * pref * s


def _u_real(l):
    U = np.zeros((2 * l + 1, 2 * l + 1), dtype=complex)
    s2 = math.sqrt(2.0)
    for m in range(-l, l + 1):
        if m == 0:
            U[l, l] = 1.0
        elif m > 0:
            U[l + m, l - m] = 1.0 / s2
            U[l + m, l + m] = ((-1.0) ** m) / s2
        else:
            a = -m
            U[l + m, l - a] = 1j / s2
            U[l + m, l + a] = -1j * ((-1.0) ** a) / s2
    return U


def _wigner_3j_real(l1, l2, l3):
    C = np.zeros((2 * l1 + 1, 2 * l2 + 1, 2 * l3 + 1), dtype=complex)
    for m1 in range(-l1, l1 + 1):
        for m2 in range(-l2, l2 + 1):
            m3 = -(m1 + m2)
            if -l3 <= m3 <= l3:
                C[m1 + l1, m2 + l2, m3 + l3] = _w3j_c(l1, l2, l3, m1, m2, m3)
    T = np.einsum('am,bn,co,mno->abc', _u_real(l1), _u_real(l2), _u_real(l3), C)
    flat = T.reshape(-1)
    k = int(np.argmax(np.abs(flat)))
    if abs(flat[k]) > 0:
        ph = flat[k] / abs(flat[k])
        T = T * np.conj(ph)
    return np.real(T).astype(np.float64)


_C011 = _wigner_3j_real(0, 1, 1)
_C101 = _wigner_3j_real(1, 0, 1)
_C110 = _wigner_3j_real(1, 1, 0)
_C111 = _wigner_3j_real(1, 1, 1)
_C112 = _wigner_3j_real(1, 1, 2)

_SQ4PI = math.sqrt(4 * math.pi)
_NORM0 = _SQ4PI * math.sqrt(1.0) / math.sqrt(8 * 1 + 8 * 1)
_NORM1 = _SQ4PI * math.sqrt(3.0) / math.sqrt(8 * 1 + 8 * 3)

_DELTA3 = float(_C011[0, 0, 0])
_EPS = float(abs(_C111[0, 1, 2]))
_EPS_TERMS = [[(b, c, float(np.sign(_C111[a, b, c])))
               for b in range(3) for c in range(3)
               if abs(_C111[a, b, c]) > 1e-12] for a in range(3)]
_C112_TERMS = [[[(c, float(_C112[a, b, c])) for c in range(5)
                 if abs(_C112[a, b, c]) > 1e-12] for b in range(3)]
               for a in range(3)]

_INV_SQRT_H = 1.0 / math.sqrt(_H)
_INV_SQRT_B = 1.0 / math.sqrt(_NUM_BASIS)


def _build_w3_perm_scale():
    perm = np.zeros(_R_DIM, dtype=np.int64)
    scale = np.zeros(_R_DIM, dtype=np.float64)
    for v in range(8):
        for u in range(8):
            perm[v * 8 + u] = u * 8 + v
            scale[v * 8 + u] = _NORM0
            perm[64 + v * 8 + u] = 64 + u * 8 + v
            scale[64 + v * 8 + u] = _NORM0 * _DELTA3
            perm[128 + v * 8 + u] = 128 + u * 8 + v
            scale[128 + v * 8 + u] = _NORM1 * _DELTA3
            for fi in range(3):
                perm[192 + (v * 3 + fi) * 8 + u] = 192 + (u * 8 + v) * 3 + fi
                scale[192 + (v * 3 + fi) * 8 + u] = _NORM1 * (
                    _DELTA3 if fi == 0 else (_EPS if fi == 1 else 1.0))
    return perm, (scale * _INV_SQRT_H).astype(np.float32)


_W3_PERM, _W3_SCALE = _build_w3_perm_scale()

_F_PERM = np.concatenate([
    np.arange(8),
    np.array([8 + v * 3 + b for b in range(3) for v in range(8)]),
]).astype(np.int32)

_MSG_PERM = np.concatenate([
    np.arange(8),
    np.array([8 + a * 8 + u for u in range(8) for a in range(3)]),
]).astype(np.int32)


def _silu(z):
    return z * (1.0 / (1.0 + jnp.exp(-z)))


def _tree_sum(xs):
    xs = list(xs)
    while len(xs) > 1:
        nxt = [a + b for a, b in zip(xs[0::2], xs[1::2])]
        if len(xs) % 2:
            nxt.append(xs[-1])
        xs = nxt
    return xs[0]


def _tc_fused_body(radii_ref, yt_ref, ft_ref, w0_ref, w1_ref, w2_ref, w3_ref,
                   out_ref):
    f32 = jnp.float32
    dn = (((1,), (0,)), ((), ()))
    rrow_all = radii_ref[...]
    ft_all = ft_ref[...].T
    yt_all = yt_ref[...]
    centers = (_MIN_R + _STEP * lax.broadcasted_iota(
        jnp.int32, (_NUM_BASIS, _SUBW), 0).astype(f32))

    for sub in range(_NSUB):
        ls = slice(sub * _SUBW, (sub + 1) * _SUBW)
        rrow = rrow_all[:, ls]
        ft = ft_all[:, ls]
        yt = yt_all[:, ls]

        t = (jnp.broadcast_to(rrow, (_NUM_BASIS, _SUBW)) - centers) * (1.0 / _STEP)
        b = jnp.exp(-(t * t))
        h = _silu(lax.dot_general(w0_ref[...], b, dn, preferred_element_type=f32)
                  * _INV_SQRT_B)
        h = _silu(lax.dot_general(w1_ref[...], h, dn, preferred_element_type=f32)
                  * _INV_SQRT_H)
        h = _silu(lax.dot_general(w2_ref[...], h, dn, preferred_element_type=f32)
                  * _INV_SQRT_H)
        rt = lax.dot_general(w3_ref[...], h, dn, preferred_element_type=f32)

        f0 = ft[0:8]
        f1 = [ft[8 + bb * 8: 16 + bb * 8] for bb in range(3)]
        y0 = yt[0:1]
        y1 = [yt[1 + c: 2 + c] for c in range(3)]
        y2 = [yt[4 + c: 5 + c] for c in range(5)]

        pa = f0 * y0
        pb = f1[0] * y1[0] + (f1[1] * y1[1] + f1[2] * y1[2])
        acc0 = _tree_sum(
            [rt[v * 8: v * 8 + 8] * pa[v: v + 1] for v in range(8)]
            + [rt[64 + v * 8: 72 + v * 8] * pb[v: v + 1] for v in range(8)])
        accs = [acc0]

        for a in range(3):
            pc = f0 * y1[a]
            pd0 = f1[a] * y0
            pd1 = _tree_sum([f1[bb] * (s * y1[c])
                             for bb, c, s in _EPS_TERMS[a]])
            pd2 = _tree_sum([
                f1[bb] * _tree_sum([coeff * y2[c]
                                    for c, coeff in _C112_TERMS[a][bb]])
                for bb in range(3)])
            terms = []
            for v in range(8):
                base = 192 + v * 24
                terms += [
                    rt[128 + v * 8: 136 + v * 8] * pc[v: v + 1],
                    rt[base: base + 8] * pd0[v: v + 1],
                    rt[base + 8: base + 16] * pd1[v: v + 1],
                    rt[base + 16: base + 24] * pd2[v: v + 1],
                ]
            accs.append(_tree_sum(terms))
        out_ref[ls, :] = jnp.concatenate(accs, axis=0).T


def _tc_fused(radii2d, yt, fe, w0t, w1t, w2t, w3pt, *, interpret=False):
    const = lambda shape: pl.BlockSpec(shape, lambda i: (0, 0))
    eb = lambda rows: pl.BlockSpec((rows, _EBLK), lambda i: (0, i))
    erow = pl.BlockSpec((_EBLK, _D_IN), lambda i: (i, 0))
    return pl.pallas_call(
        _tc_fused_body,
        grid=(_N_EBLK,),
        in_specs=[
            eb(1),
            eb(9),
            erow,
            const((_H, _NUM_BASIS)),
            const((_H, _H)),
            const((_H, _H)),
            const((_R_DIM, _H)),
        ],
        out_specs=pl.BlockSpec((_EBLK, _D_OUT), lambda i: (i, 0)),
        out_shape=jax.ShapeDtypeStruct((_E_PAD, _D_OUT), jnp.float32),
        interpret=interpret,
    )(radii2d, yt, fe, w0t, w1t, w2t, w3pt)


_NW = 32
_CHUNK = 128
_CHUNKS_PER_W = 40
_E_PAD = _NW * _CHUNKS_PER_W * _CHUNK
_E_PER_W = _E_PAD // _NW
_IDX_ROWS = _E_PAD // _CHUNK
_N_ACC = 10240
_NODES_PER_W = _N_ACC // 16


_SC_PARAMS = pltpu.CompilerParams(use_tc_tiling_on_sc=False)
_NBUF = 4


def _sc_gather(xp, src2d):
    mesh = plsc.VectorSubcoreMesh(core_axis_name="c", subcore_axis_name="s")

    @functools.partial(
        pl.kernel,
        out_type=jax.ShapeDtypeStruct((_E_PAD, _D_IN), jnp.float32),
        mesh=mesh,
        scratch_types=[
            pltpu.VMEM((_CHUNKS_PER_W, _CHUNK), jnp.int32),
            pltpu.VMEM((_NBUF, _CHUNK, _D_IN), jnp.float32),
        ] + [pltpu.SemaphoreType.DMA] * _NBUF,
        compiler_params=_SC_PARAMS,
    )
    def k(x_hbm, idx_hbm, out_hbm, idx_v, bufs, *sems):
        wid = lax.axis_index("c") * 16 + lax.axis_index("s")
        pltpu.sync_copy(
            idx_hbm.at[pl.ds(wid * _CHUNKS_PER_W, _CHUNKS_PER_W)], idx_v)

        def body(t, carry):
            hs = []
            for p in range(_NBUF):
                j = t * _NBUF + p
                hs.append(pltpu.async_copy(
                    x_hbm.at[idx_v.at[j]], bufs.at[p], sems[p]))
            for p in range(_NBUF):
                j = t * _NBUF + p
                hs[p].wait()
                pltpu.sync_copy(
                    bufs.at[p],
                    out_hbm.at[pl.ds(wid * _E_PER_W + j * _CHUNK, _CHUNK)])
            return carry

        lax.fori_loop(0, _CHUNKS_PER_W // _NBUF, body, 0)

    return k(xp, src2d)


def _sc_scatter(msg, dst2d, zrows):
    mesh = plsc.VectorSubcoreMesh(core_axis_name="c", subcore_axis_name="s")

    @functools.partial(
        pl.kernel,
        out_type=jax.ShapeDtypeStruct((2, _N_ACC, _D_OUT), jnp.float32),
        mesh=mesh,
        scratch_types=[
            pltpu.VMEM((_CHUNKS_PER_W, _CHUNK), jnp.int32),
            pltpu.VMEM((_NBUF, _CHUNK, _D_OUT), jnp.float32),
            pltpu.VMEM_SHARED((_N_ACC, _D_OUT), jnp.float32),
        ] + [pltpu.SemaphoreType.DMA] * _NBUF,
        compiler_params=_SC_PARAMS,
    )
    def k(msg_hbm, idx_hbm, z_hbm, out_hbm, idx_v, bufs, acc_sh, *sems):
        c = lax.axis_index("c")
        s = lax.axis_index("s")
        wid = c * 16 + s
        pltpu.sync_copy(z_hbm, acc_sh.at[pl.ds(s * _NODES_PER_W, _NODES_PER_W)])
        pltpu.sync_copy(
            idx_hbm.at[pl.ds(wid * _CHUNKS_PER_W, _CHUNKS_PER_W)], idx_v)
        plsc.subcore_barrier()

        def body(t, carry):
            hs = []
            for p in range(_NBUF):
                j = t * _NBUF + p
                hs.append(pltpu.async_copy(
                    msg_hbm.at[pl.ds(wid * _E_PER_W + j * _CHUNK, _CHUNK)],
                    bufs.at[p], sems[p]))
            for p in range(_NBUF):
                j = t * _NBUF + p
                hs[p].wait()
                pltpu.sync_copy(bufs.at[p], acc_sh.at[idx_v.at[j]], add=True)
            return carry

        lax.fori_loop(0, _CHUNKS_PER_W // _NBUF, body, 0)
        plsc.subcore_barrier()
        pltpu.sync_copy(
            acc_sh.at[pl.ds(s * _NODES_PER_W, _NODES_PER_W)],
            out_hbm.at[c, pl.ds(s * _NODES_PER_W, _NODES_PER_W)])

    return k(msg, dst2d, zrows)


@jax.jit
def kernel(x, edge_index, radii, rsh, W0, W1, W2, W3):
    src = edge_index[0].astype(jnp.int32)
    dst = edge_index[1].astype(jnp.int32)

    npad = _E_PAD - _N_EDGES
    xp = x[:, _F_PERM]
    src_pad = jnp.concatenate([src, jnp.zeros((npad,), jnp.int32)])
    f_e = _sc_gather(xp, src_pad.reshape(_IDX_ROWS, _CHUNK))
    yt = rsh.T
    radii2d = radii.reshape(1, _N_EDGES)
    w3pt = (W3[:, _W3_PERM] * _W3_SCALE[None, :]).T

    msg_pad = _tc_fused(radii2d, yt, f_e, W0.T, W1.T, W2.T, w3pt)
    dst_pad = jnp.concatenate([dst, jnp.full((npad,), _N_NODES, jnp.int32)])

    zrows = jnp.zeros((_NODES_PER_W, _D_OUT), jnp.float32)
    parts = _sc_scatter(msg_pad, dst_pad.reshape(_IDX_ROWS, _CHUNK), zrows)
    return (parts[0, :_N_NODES] + parts[1, :_N_NODES])[:, _MSG_PERM]
